# baseline (speedup 1.0000x reference)
import jax
import jax.numpy as jnp
from jax.experimental import pallas as pl

NUM_CLASSES = 1000
ROWS = 4096 * 20
BLOCK_R = 256
NUM_BLOCKS = ROWS // BLOCK_R


def _one_hot_kernel(idx_ref, out_ref):
    idx = idx_ref[0, 0, :]
    classes = jax.lax.broadcasted_iota(jnp.int32, (BLOCK_R, NUM_CLASSES), 1)
    out_ref[...] = (idx[:, None] == classes).astype(jnp.float32)


def kernel(inputs):
    idx = inputs.reshape(NUM_BLOCKS, 1, BLOCK_R).astype(jnp.int32)
    out = pl.pallas_call(
        _one_hot_kernel,
        grid=(NUM_BLOCKS,),
        in_specs=[pl.BlockSpec((1, 1, BLOCK_R), lambda i: (i, 0, 0))],
        out_specs=pl.BlockSpec((BLOCK_R, NUM_CLASSES), lambda i: (i, 0)),
        out_shape=jax.ShapeDtypeStruct((ROWS, NUM_CLASSES), jnp.float32),
    )(idx)
    return out.reshape(4096, 20, NUM_CLASSES)


# speedup vs baseline: 1.1532x; 1.1532x over previous
import jax
import jax.numpy as jnp
from jax.experimental import pallas as pl

NUM_CLASSES = 1000
ROWS = 4096 * 20
BLOCK_R = 2048
NUM_BLOCKS = ROWS // BLOCK_R


def _one_hot_kernel(idx_ref, out_ref):
    idx = idx_ref[0, 0, :]
    classes = jax.lax.broadcasted_iota(jnp.int32, (BLOCK_R, NUM_CLASSES), 1)
    out_ref[...] = (idx[:, None] == classes).astype(jnp.float32)


def kernel(inputs):
    idx = inputs.reshape(NUM_BLOCKS, 1, BLOCK_R).astype(jnp.int32)
    out = pl.pallas_call(
        _one_hot_kernel,
        grid=(NUM_BLOCKS,),
        in_specs=[pl.BlockSpec((1, 1, BLOCK_R), lambda i: (i, 0, 0))],
        out_specs=pl.BlockSpec((BLOCK_R, NUM_CLASSES), lambda i: (i, 0)),
        out_shape=jax.ShapeDtypeStruct((ROWS, NUM_CLASSES), jnp.float32),
    )(idx)
    return out.reshape(4096, 20, NUM_CLASSES)
